# baseline (speedup 1.0000x reference)
import functools

import jax
import jax.numpy as jnp
from jax import lax
from jax.experimental import pallas as pl
from jax.experimental.pallas import tpu as pltpu
from jax.experimental.pallas import tpu_sc as plsc

PAD = 1
B_ROWS = 4
SEQ = 2048
EMB = 1024
TOKENS = B_ROWS * SEQ

NC = 2
NS = 16
NW = NC * NS
CHUNK = TOKENS // NW
CHUNKS_PER_ROW = SEQ // CHUNK
LANES = 16
NBUF = 3
BMAX = 32

BS = (8, 24, 32, 32, 32, 32, 32, 32, 16, 16)
OFFS = tuple(sum(BS[:i]) for i in range(len(BS)))
assert sum(BS) == CHUNK and all(o % 8 == 0 for o in OFFS)


def _sc_kernel(tok_hbm, w_hbm, out_hbm, tok_v, pos_v, shf_v, buf0, buf1,
               buf2, gsem0, gsem1, gsem2, osem0, osem1, osem2):
  wid = lax.axis_index("s") * NC + lax.axis_index("c")
  row = wid // CHUNKS_PER_ROW
  chunk = wid % CHUNKS_PER_ROW
  t0 = chunk * CHUNK
  base = row * SEQ + t0

  pltpu.sync_copy(tok_hbm.at[pl.ds(row * SEQ, SEQ)], tok_v)

  shf_v[pl.ds(0, LANES)] = jnp.zeros((LANES,), jnp.int32)

  def cumsum16(x):
    for k in (1, 2, 4, 8):
      shf_v[pl.ds(LANES, LANES)] = x
      x = x + shf_v[pl.ds(LANES - k, LANES)]
    return x

  def mask16(v):
    return jnp.minimum(jnp.abs(v - PAD), 1)

  def count_body(jj, accs):
    a0, a1, a2, a3 = accs
    b = jj * (4 * LANES)
    a0 = a0 + mask16(tok_v[pl.ds(b, LANES)])
    a1 = a1 + mask16(tok_v[pl.ds(b + LANES, LANES)])
    a2 = a2 + mask16(tok_v[pl.ds(b + 2 * LANES, LANES)])
    a3 = a3 + mask16(tok_v[pl.ds(b + 3 * LANES, LANES)])
    return a0, a1, a2, a3

  zeros = jnp.zeros((LANES,), jnp.int32)
  a0, a1, a2, a3 = lax.fori_loop(0, t0 // (4 * LANES), count_body,
                                 (zeros, zeros, zeros, zeros))
  carry = cumsum16((a0 + a1) + (a2 + a3))[LANES - 1]

  bufs = (buf0, buf1, buf2)
  gsems = (gsem0, gsem1, gsem2)
  osems = (osem0, osem1, osem2)
  NSUB = len(BS)

  def gather(i):
    b = i % NBUF
    return pltpu.async_copy(w_hbm.at[pos_v.at[pl.ds(OFFS[i], BS[i])]],
                            bufs[b].at[pl.ds(0, BS[i])], gsems[b])

  def put(i):
    b = i % NBUF
    return pltpu.async_copy(bufs[b].at[pl.ds(0, BS[i])],
                            out_hbm.at[pl.ds(base + OFFS[i], BS[i])],
                            osems[b])

  gh = [None] * NSUB
  oh = [None] * NSUB
  waited = set()
  g_done = 0
  for i in range(NSUB):
    need = (OFFS[i] + BS[i] + LANES - 1) // LANES
    while g_done < need:
      v = tok_v[pl.ds(t0 + g_done * LANES, LANES)]
      mi = mask16(v)
      cs = cumsum16(mi)
      pos_v[pl.ds(g_done * LANES, LANES)] = (carry + cs) * mi + 1
      carry = carry + cs[LANES - 1]
      g_done += 1
    if i >= NBUF:
      oh[i - NBUF].wait()
      waited.add(i - NBUF)
    gh[i] = gather(i)
    if i >= 1:
      gh[i - 1].wait()
      oh[i - 1] = put(i - 1)
  gh[NSUB - 1].wait()
  oh[NSUB - 1] = put(NSUB - 1)
  for i in range(NSUB):
    if i not in waited:
      oh[i].wait()


@jax.jit
def _lookup(tokens_flat, w):
  mesh = plsc.VectorSubcoreMesh(core_axis_name="c", subcore_axis_name="s")
  k = functools.partial(
      pl.kernel,
      mesh=mesh,
      out_type=jax.ShapeDtypeStruct((TOKENS, EMB), jnp.float32),
      scratch_types=[
          pltpu.VMEM((SEQ,), jnp.int32),
          pltpu.VMEM((CHUNK,), jnp.int32),
          pltpu.VMEM((2 * LANES,), jnp.int32),
          pltpu.VMEM((BMAX, EMB), jnp.float32),
          pltpu.VMEM((BMAX, EMB), jnp.float32),
          pltpu.VMEM((BMAX, EMB), jnp.float32),
          pltpu.SemaphoreType.DMA,
          pltpu.SemaphoreType.DMA,
          pltpu.SemaphoreType.DMA,
          pltpu.SemaphoreType.DMA,
          pltpu.SemaphoreType.DMA,
          pltpu.SemaphoreType.DMA,
      ],
  )(_sc_kernel)
  return k(tokens_flat, w)


def kernel(tokens, W):
  tokens_flat = tokens.astype(jnp.int32).reshape(TOKENS)
  out = _lookup(tokens_flat, W)
  return out.reshape(B_ROWS, SEQ, EMB)


# speedup vs baseline: 1.0213x; 1.0213x over previous
import functools

import jax
import jax.numpy as jnp
from jax import lax
from jax.experimental import pallas as pl
from jax.experimental.pallas import tpu as pltpu
from jax.experimental.pallas import tpu_sc as plsc

PAD = 1
B_ROWS = 4
SEQ = 2048
EMB = 1024
TOKENS = B_ROWS * SEQ

NC = 2
NS = 16
NW = NC * NS
CHUNK = TOKENS // NW
CHUNKS_PER_ROW = SEQ // CHUNK
LANES = 16
NBUF = 3
BMAX = 32

BS = (32,) * 8
OFFS = tuple(sum(BS[:i]) for i in range(len(BS)))
assert sum(BS) == CHUNK and all(o % 8 == 0 for o in OFFS)


def _sc_kernel(tok_hbm, w_hbm, out_hbm, tok_v, pos_v, shf_v, buf0, buf1,
               buf2, gsem0, gsem1, gsem2, osem0, osem1, osem2):
  wid = lax.axis_index("s") * NC + lax.axis_index("c")
  row = wid // CHUNKS_PER_ROW
  chunk = wid % CHUNKS_PER_ROW
  t0 = chunk * CHUNK
  base = row * SEQ + t0

  pltpu.sync_copy(tok_hbm.at[pl.ds(row * SEQ, SEQ)], tok_v)

  shf_v[pl.ds(0, LANES)] = jnp.zeros((LANES,), jnp.int32)

  def cumsum16(x):
    for k in (1, 2, 4, 8):
      shf_v[pl.ds(LANES, LANES)] = x
      x = x + shf_v[pl.ds(LANES - k, LANES)]
    return x

  def mask16(v):
    return jnp.minimum(jnp.abs(v - PAD), 1)

  def count_body(jj, accs):
    a0, a1, a2, a3 = accs
    b = jj * (4 * LANES)
    a0 = a0 + mask16(tok_v[pl.ds(b, LANES)])
    a1 = a1 + mask16(tok_v[pl.ds(b + LANES, LANES)])
    a2 = a2 + mask16(tok_v[pl.ds(b + 2 * LANES, LANES)])
    a3 = a3 + mask16(tok_v[pl.ds(b + 3 * LANES, LANES)])
    return a0, a1, a2, a3

  zeros = jnp.zeros((LANES,), jnp.int32)
  a0, a1, a2, a3 = lax.fori_loop(0, t0 // (4 * LANES), count_body,
                                 (zeros, zeros, zeros, zeros))
  carry = cumsum16((a0 + a1) + (a2 + a3))[LANES - 1]

  bufs = (buf0, buf1, buf2)
  gsems = (gsem0, gsem1, gsem2)
  osems = (osem0, osem1, osem2)
  NSUB = len(BS)

  def gather(i):
    b = i % NBUF
    return pltpu.async_copy(w_hbm.at[pos_v.at[pl.ds(OFFS[i], BS[i])]],
                            bufs[b].at[pl.ds(0, BS[i])], gsems[b])

  def put(i):
    b = i % NBUF
    return pltpu.async_copy(bufs[b].at[pl.ds(0, BS[i])],
                            out_hbm.at[pl.ds(base + OFFS[i], BS[i])],
                            osems[b])

  gh = [None] * NSUB
  oh = [None] * NSUB
  waited = set()
  g_done = 0
  for i in range(NSUB):
    need = (OFFS[i] + BS[i] + LANES - 1) // LANES
    while g_done < need:
      v = tok_v[pl.ds(t0 + g_done * LANES, LANES)]
      mi = mask16(v)
      cs = cumsum16(mi)
      pos_v[pl.ds(g_done * LANES, LANES)] = (carry + cs) * mi + 1
      carry = carry + cs[LANES - 1]
      g_done += 1
    if i >= NBUF:
      oh[i - NBUF].wait()
      waited.add(i - NBUF)
    gh[i] = gather(i)
    if i >= 1:
      gh[i - 1].wait()
      oh[i - 1] = put(i - 1)
  gh[NSUB - 1].wait()
  oh[NSUB - 1] = put(NSUB - 1)
  for i in range(NSUB):
    if i not in waited:
      oh[i].wait()


@jax.jit
def _lookup(tokens_flat, w):
  mesh = plsc.VectorSubcoreMesh(core_axis_name="c", subcore_axis_name="s")
  k = functools.partial(
      pl.kernel,
      mesh=mesh,
      out_type=jax.ShapeDtypeStruct((TOKENS, EMB), jnp.float32),
      scratch_types=[
          pltpu.VMEM((SEQ,), jnp.int32),
          pltpu.VMEM((CHUNK,), jnp.int32),
          pltpu.VMEM((2 * LANES,), jnp.int32),
          pltpu.VMEM((BMAX, EMB), jnp.float32),
          pltpu.VMEM((BMAX, EMB), jnp.float32),
          pltpu.VMEM((BMAX, EMB), jnp.float32),
          pltpu.SemaphoreType.DMA,
          pltpu.SemaphoreType.DMA,
          pltpu.SemaphoreType.DMA,
          pltpu.SemaphoreType.DMA,
          pltpu.SemaphoreType.DMA,
          pltpu.SemaphoreType.DMA,
      ],
  )(_sc_kernel)
  return k(tokens_flat, w)


def kernel(tokens, W):
  tokens_flat = tokens.astype(jnp.int32).reshape(TOKENS)
  out = _lookup(tokens_flat, W)
  return out.reshape(B_ROWS, SEQ, EMB)
